# baseline (speedup 1.0000x reference)
import functools
import math

import jax
import jax.numpy as jnp
from jax.experimental import pallas as pl
from jax.experimental.pallas import tpu as pltpu

T = 2048
D = 1024
F = 4096
E = 8
K = 2
CAP = int(T * K / E * 1.25)

NF = 4
FC = F // NF
S = 128
NCH = T // S
W = S + 8


def _router_kernel(x_ref, ls_ref, lb_ref, rw_ref, h_ref, ei_ref, gv_ref):
    x = x_ref[...]
    mu = jnp.mean(x, axis=-1, keepdims=True)
    xc = x - mu
    var = jnp.mean(xc * xc, axis=-1, keepdims=True)
    h = xc / jnp.sqrt(var + 1e-5) * ls_ref[...][None, :] + lb_ref[...][None, :]
    h_ref[...] = h
    logits = jnp.dot(h, rw_ref[...], preferred_element_type=jnp.float32)
    mx = jnp.max(logits, axis=-1, keepdims=True)
    ex = jnp.exp(logits - mx)
    probs = ex / jnp.sum(ex, axis=-1, keepdims=True)
    iota = jax.lax.broadcasted_iota(jnp.int32, probs.shape, 1)
    v1 = jnp.max(probs, axis=-1, keepdims=True)
    i1 = jnp.min(jnp.where(probs == v1, iota, E), axis=-1, keepdims=True)
    masked = jnp.where(iota == i1, -jnp.inf, probs)
    v2 = jnp.max(masked, axis=-1, keepdims=True)
    i2 = jnp.min(jnp.where(masked == v2, iota, E), axis=-1, keepdims=True)
    s = v1 + v2 + 1e-8
    ei_ref[...] = jnp.concatenate([i1, i2], axis=-1)
    gv_ref[...] = jnp.concatenate([v1 / s, v2 / s], axis=-1)


def _pos_kernel(ei_ref, gv_ref, pos_ref, w_ref, st_ref):
    ei = ei_ref[...]
    gv = gv_ref[...]
    eiota = jax.lax.broadcasted_iota(jnp.int32, (T, E), 1)
    c = ((ei[:, 0:1] == eiota).astype(jnp.float32)
         + (ei[:, 1:2] == eiota).astype(jnp.float32))
    r = jax.lax.broadcasted_iota(jnp.int32, (T, T), 0)
    col = jax.lax.broadcasted_iota(jnp.int32, (T, T), 1)
    L = (col < r).astype(jnp.float32)
    excl = jax.lax.dot_general(
        L, c, (((1,), (0,)), ((), ())),
        preferred_element_type=jnp.float32,
        precision=jax.lax.Precision.HIGHEST)
    eoh0 = (ei[:, 0:1] == eiota).astype(jnp.float32)
    eoh1 = (ei[:, 1:2] == eiota).astype(jnp.float32)
    pos0 = jnp.sum(excl * eoh0, axis=-1, keepdims=True)
    pos1 = jnp.sum(excl * eoh1, axis=-1, keepdims=True)
    pos = jnp.concatenate([pos0, pos1], axis=-1)
    keep = (pos < CAP).astype(jnp.float32)
    pos_c = jnp.minimum(pos, CAP - 1).astype(jnp.int32)
    pos_ref[...] = pos_c
    w_ref[...] = gv * keep
    rj = jax.lax.broadcasted_iota(jnp.int32, (NCH, T), 0)
    ct = jax.lax.broadcasted_iota(jnp.int32, (NCH, T), 1)
    Lc = (ct < rj * S).astype(jnp.float32)
    st = jax.lax.dot_general(
        Lc, c, (((1,), (0,)), ((), ())),
        preferred_element_type=jnp.float32,
        precision=jax.lax.Precision.HIGHEST)
    st_ref[...] = st.astype(jnp.int32)


def _chunk_masks(ei_j, pos_j, w_j, e, start):
    citer = jax.lax.broadcasted_iota(jnp.int32, (S, W), 1)
    sel0 = (ei_j[:, 0:1] == e) & (w_j[:, 0:1] > 0.0)
    sel1 = (ei_j[:, 1:2] == e) & (w_j[:, 1:2] > 0.0)
    q0 = jnp.where(sel0, pos_j[:, 0:1] - start, -1)
    q1 = jnp.where(sel1, pos_j[:, 1:2] - start, -1)
    m0 = q0 == citer
    m1 = q1 == citer
    M = (m0.astype(jnp.float32) + m1.astype(jnp.float32)).astype(jnp.bfloat16)
    C = (jnp.where(m0, w_j[:, 0:1], 0.0)
         + jnp.where(m1, w_j[:, 1:2], 0.0)).astype(jnp.bfloat16)
    return M, C


def _moe_kernel(st_ref, h_ref, ei_ref, pos_ref, w_ref, w1_ref, b1_ref,
                w2_ref, b2_ref, y_ref, disp_ref, acc_ref):
    e = pl.program_id(0)
    f = pl.program_id(1)

    @pl.when(f == 0)
    def _dispatch():
        disp_ref[...] = jnp.zeros((CAP + W, D), jnp.bfloat16)

        def body(j, carry):
            start = pl.multiple_of(
                (jnp.minimum(st_ref[j, e], CAP) // 8) * 8, 8)
            ei_j = ei_ref[pl.ds(j * S, S), :]
            pos_j = pos_ref[pl.ds(j * S, S), :]
            w_j = w_ref[pl.ds(j * S, S), :]
            M, _ = _chunk_masks(ei_j, pos_j, w_j, e, start)
            h_j = h_ref[pl.ds(j * S, S), :].astype(jnp.bfloat16)
            r = jax.lax.dot_general(
                M, h_j, (((0,), (0,)), ((), ())),
                preferred_element_type=jnp.float32)
            disp_ref[pl.ds(start, W), :] = (
                disp_ref[pl.ds(start, W), :] + r.astype(jnp.bfloat16))
            return carry

        jax.lax.fori_loop(0, NCH, body, 0)

    hidden = jnp.maximum(
        jnp.dot(disp_ref[0:CAP], w1_ref[0].astype(jnp.bfloat16),
                preferred_element_type=jnp.float32)
        + b1_ref[0], 0.0).astype(jnp.bfloat16)
    part = jnp.dot(hidden, w2_ref[0].astype(jnp.bfloat16),
                   preferred_element_type=jnp.float32)

    @pl.when(f == 0)
    def _init_acc():
        acc_ref[...] = part

    @pl.when((f != 0) & (f != NF - 1))
    def _add_acc():
        acc_ref[...] = acc_ref[...] + part

    @pl.when(f == NF - 1)
    def _combine():
        eout = acc_ref[...] + part + b2_ref[0]
        disp_ref[0:CAP] = eout.astype(jnp.bfloat16)

        @pl.when(e == 0)
        def _():
            y_ref[...] = jnp.zeros((T, D), jnp.float32)

        def body(j, carry):
            start = pl.multiple_of(
                (jnp.minimum(st_ref[j, e], CAP) // 8) * 8, 8)
            ei_j = ei_ref[pl.ds(j * S, S), :]
            pos_j = pos_ref[pl.ds(j * S, S), :]
            w_j = w_ref[pl.ds(j * S, S), :]
            _, C = _chunk_masks(ei_j, pos_j, w_j, e, start)
            eo_j = disp_ref[pl.ds(start, W), :]
            yp = jnp.dot(C, eo_j, preferred_element_type=jnp.float32)
            y_ref[pl.ds(j * S, S), :] = y_ref[pl.ds(j * S, S), :] + yp
            return carry

        jax.lax.fori_loop(0, NCH, body, 0)


def kernel(x, ln_scale, ln_bias, router_w, w1, b1, w2, b2):
    h, ei, gv = pl.pallas_call(
        _router_kernel,
        out_shape=[
            jax.ShapeDtypeStruct((T, D), jnp.float32),
            jax.ShapeDtypeStruct((T, K), jnp.int32),
            jax.ShapeDtypeStruct((T, K), jnp.float32),
        ],
    )(x, ln_scale, ln_bias, router_w)

    pos, w, starts = pl.pallas_call(
        _pos_kernel,
        out_shape=[
            jax.ShapeDtypeStruct((T, K), jnp.int32),
            jax.ShapeDtypeStruct((T, K), jnp.float32),
            jax.ShapeDtypeStruct((NCH, E), jnp.int32),
        ],
    )(ei, gv)

    y = pl.pallas_call(
        _moe_kernel,
        grid_spec=pltpu.PrefetchScalarGridSpec(
            num_scalar_prefetch=1,
            grid=(E, NF),
            in_specs=[
                pl.BlockSpec((T, D), lambda e, f, st: (0, 0)),
                pl.BlockSpec((T, K), lambda e, f, st: (0, 0)),
                pl.BlockSpec((T, K), lambda e, f, st: (0, 0)),
                pl.BlockSpec((T, K), lambda e, f, st: (0, 0)),
                pl.BlockSpec((1, D, FC), lambda e, f, st: (e, 0, f)),
                pl.BlockSpec((1, 1, FC), lambda e, f, st: (e, 0, f)),
                pl.BlockSpec((1, FC, D), lambda e, f, st: (e, f, 0)),
                pl.BlockSpec((1, 1, D), lambda e, f, st: (e, 0, 0)),
            ],
            out_specs=pl.BlockSpec((T, D), lambda e, f, st: (0, 0)),
            scratch_shapes=[
                pltpu.VMEM((CAP + W, D), jnp.bfloat16),
                pltpu.VMEM((CAP, D), jnp.float32),
            ],
        ),
        out_shape=jax.ShapeDtypeStruct((T, D), jnp.float32),
    )(starts, h, ei, pos, w, w1, b1.reshape(E, 1, F), w2,
      b2.reshape(E, 1, D))
    return y


# speedup vs baseline: 1.0518x; 1.0518x over previous
import functools

import jax
import jax.numpy as jnp
from jax import lax
from jax.experimental import pallas as pl
from jax.experimental.pallas import tpu as pltpu
from jax.experimental.pallas import tpu_sc as plsc

T = 2048
D = 1024
F = 4096
E = 8
K = 2
CAP = int(T * K / E * 1.25)

NF = 4
FC = F // NF

NSLOT = E * CAP
TRASH = NSLOT
NROW = NSLOT + 8

NW = 32
TPW = T // NW
CH = 32


def _router_kernel(x_ref, ls_ref, lb_ref, rw_ref, h_ref, ei_ref, gv_ref):
    x = x_ref[...]
    mu = jnp.mean(x, axis=-1, keepdims=True)
    xc = x - mu
    var = jnp.mean(xc * xc, axis=-1, keepdims=True)
    h = xc / jnp.sqrt(var + 1e-5) * ls_ref[...][None, :] + lb_ref[...][None, :]
    h_ref[...] = h
    logits = jnp.dot(h, rw_ref[...], preferred_element_type=jnp.float32)
    mx = jnp.max(logits, axis=-1, keepdims=True)
    ex = jnp.exp(logits - mx)
    probs = ex / jnp.sum(ex, axis=-1, keepdims=True)
    iota = jax.lax.broadcasted_iota(jnp.int32, probs.shape, 1)
    v1 = jnp.max(probs, axis=-1, keepdims=True)
    i1 = jnp.min(jnp.where(probs == v1, iota, E), axis=-1, keepdims=True)
    masked = jnp.where(iota == i1, -jnp.inf, probs)
    v2 = jnp.max(masked, axis=-1, keepdims=True)
    i2 = jnp.min(jnp.where(masked == v2, iota, E), axis=-1, keepdims=True)
    s = v1 + v2 + 1e-8
    ei_ref[...] = jnp.concatenate([i1, i2], axis=-1)
    gv_ref[...] = jnp.concatenate([v1 / s, v2 / s], axis=-1)


def _pos_kernel(ei_ref, gv_ref, s0_ref, s1_ref, g0_ref, g1_ref, sw_ref):
    ei = ei_ref[...]
    gv = gv_ref[...]
    eiota = jax.lax.broadcasted_iota(jnp.int32, (T, E), 1)
    eoh0 = (ei[:, 0:1] == eiota).astype(jnp.float32)
    eoh1 = (ei[:, 1:2] == eiota).astype(jnp.float32)
    c = eoh0 + eoh1
    r = jax.lax.broadcasted_iota(jnp.int32, (T, T), 0)
    col = jax.lax.broadcasted_iota(jnp.int32, (T, T), 1)
    L = (col < r).astype(jnp.float32)
    excl = jax.lax.dot_general(
        L, c, (((1,), (0,)), ((), ())),
        preferred_element_type=jnp.float32,
        precision=jax.lax.Precision.HIGHEST)
    pos0 = jnp.sum(excl * eoh0, axis=-1, keepdims=True)
    pos1 = jnp.sum(excl * eoh1, axis=-1, keepdims=True)
    keep0 = pos0 < CAP
    keep1 = pos1 < CAP
    slot0 = ei[:, 0:1] * CAP + pos0.astype(jnp.int32)
    slot1 = ei[:, 1:2] * CAP + pos1.astype(jnp.int32)
    s0_ref[...] = jnp.where(keep0, slot0, TRASH)
    s1_ref[...] = jnp.where(keep1, slot1, TRASH)

    tot = excl[T - 1:T, :] + c[T - 1:T, :]
    tmin = jnp.min(tot, axis=-1, keepdims=True)
    e_iota_row = jax.lax.broadcasted_iota(jnp.int32, (1, E), 1)
    pe = jnp.min(jnp.where(tot == tmin, e_iota_row, E), axis=-1,
                 keepdims=True)
    pad_slot = pe * CAP + (CAP - 1)
    g0_ref[...] = jnp.where(keep0, slot0, pad_slot)
    g1_ref[...] = jnp.where(keep1, slot1, pad_slot)

    ident = (r == col).astype(jnp.float32)
    p01 = jnp.concatenate([pos0, pos1], axis=-1)
    p01_row = jax.lax.dot_general(
        p01, ident, (((0,), (0,)), ((), ())),
        preferred_element_type=jnp.float32,
        precision=jax.lax.Precision.HIGHEST)
    p01i = p01_row.astype(jnp.int32)
    citer = jax.lax.broadcasted_iota(jnp.int32, (CAP, T), 0)
    P0 = p01i[0:1, :] == citer
    P1 = p01i[1:2, :] == citer
    w = gv * jnp.concatenate(
        [keep0.astype(jnp.float32), keep1.astype(jnp.float32)], axis=-1)
    W0 = eoh0 * w[:, 0:1]
    W1 = eoh1 * w[:, 1:2]
    sw_ref[...] = (
        jnp.dot(P0.astype(jnp.float32), W0, preferred_element_type=jnp.float32,
                precision=jax.lax.Precision.HIGHEST)
        + jnp.dot(P1.astype(jnp.float32), W1, preferred_element_type=jnp.float32,
                  precision=jax.lax.Precision.HIGHEST))


def _sc_dispatch(h_hbm, s0_hbm, s1_hbm, disp_hbm, rows_v, i0_v, i1_v, sem):
    wid = lax.axis_index("s") * 2 + lax.axis_index("c")
    base = wid * TPW
    pltpu.sync_copy(h_hbm.at[pl.ds(base, TPW)], rows_v)
    pltpu.sync_copy(s0_hbm.at[pl.ds(base, TPW)], i0_v)
    pltpu.sync_copy(s1_hbm.at[pl.ds(base, TPW)], i1_v)
    pltpu.async_copy(rows_v, disp_hbm.at[i0_v], sem).wait()
    pltpu.async_copy(rows_v, disp_hbm.at[i1_v], sem).wait()


def _sc_combine(eout_hbm, g0_hbm, g1_hbm, y_hbm, a_v, b_v, g0_v, g1_v, sem):
    wid = lax.axis_index("s") * 2 + lax.axis_index("c")
    base = wid * TPW
    for sub in range(TPW // CH):
        off = base + sub * CH
        pltpu.sync_copy(g0_hbm.at[pl.ds(off, CH)], g0_v)
        pltpu.sync_copy(g1_hbm.at[pl.ds(off, CH)], g1_v)
        pltpu.async_copy(eout_hbm.at[g0_v], a_v, sem).wait()
        pltpu.async_copy(eout_hbm.at[g1_v], b_v, sem).wait()

        def row_body(rr, carry):
            for cc in range(0, D, 16):
                a_v[rr, pl.ds(cc, 16)] = (a_v[rr, pl.ds(cc, 16)]
                                          + b_v[rr, pl.ds(cc, 16)])
            return carry

        lax.fori_loop(0, CH, row_body, 0)
        pltpu.sync_copy(a_v, y_hbm.at[pl.ds(off, CH)])


def _moe_kernel(disp_ref, w1_ref, b1_ref, w2_ref, b2_ref, sw_ref,
                eout_ref, dispb_ref, acc_ref):
    f = pl.program_id(1)

    @pl.when(f == 0)
    def _sanitize():
        d = disp_ref[...]
        dispb_ref[...] = jnp.where(jnp.isfinite(d), d, 0.0).astype(jnp.bfloat16)

    hidden = jnp.maximum(
        jnp.dot(dispb_ref[...], w1_ref[0].astype(jnp.bfloat16),
                preferred_element_type=jnp.float32)
        + b1_ref[0], 0.0).astype(jnp.bfloat16)
    part = jnp.dot(hidden, w2_ref[0].astype(jnp.bfloat16),
                   preferred_element_type=jnp.float32)

    @pl.when(f == 0)
    def _init_acc():
        acc_ref[...] = part

    @pl.when((f != 0) & (f != NF - 1))
    def _add_acc():
        acc_ref[...] = acc_ref[...] + part

    @pl.when(f == NF - 1)
    def _scale_out():
        e = pl.program_id(0)
        laneiota = jax.lax.broadcasted_iota(jnp.int32, (CAP, E), 1)
        swc = jnp.sum(jnp.where(laneiota == e, sw_ref[...], 0.0),
                      axis=-1, keepdims=True)
        eout_ref[...] = (acc_ref[...] + part + b2_ref[0]) * swc


def kernel(x, ln_scale, ln_bias, router_w, w1, b1, w2, b2):
    h, ei, gv = pl.pallas_call(
        _router_kernel,
        out_shape=[
            jax.ShapeDtypeStruct((T, D), jnp.float32),
            jax.ShapeDtypeStruct((T, K), jnp.int32),
            jax.ShapeDtypeStruct((T, K), jnp.float32),
        ],
    )(x, ln_scale, ln_bias, router_w)

    s0, s1, g0, g1, sw = pl.pallas_call(
        _pos_kernel,
        out_shape=[
            jax.ShapeDtypeStruct((T, 1), jnp.int32),
            jax.ShapeDtypeStruct((T, 1), jnp.int32),
            jax.ShapeDtypeStruct((T, 1), jnp.int32),
            jax.ShapeDtypeStruct((T, 1), jnp.int32),
            jax.ShapeDtypeStruct((CAP, E), jnp.float32),
        ],
    )(ei, gv)

    mesh = plsc.VectorSubcoreMesh(core_axis_name="c", subcore_axis_name="s")

    disp = pl.kernel(
        _sc_dispatch,
        mesh=mesh,
        out_type=jax.ShapeDtypeStruct((NROW, D), jnp.float32),
        scratch_types=[
            pltpu.VMEM((TPW, D), jnp.float32),
            pltpu.VMEM((TPW,), jnp.int32),
            pltpu.VMEM((TPW,), jnp.int32),
            pltpu.SemaphoreType.DMA,
        ],
    )(h, s0.reshape(T), s1.reshape(T))

    eout = pl.pallas_call(
        _moe_kernel,
        grid=(E, NF),
        in_specs=[
            pl.BlockSpec((CAP, D), lambda e, f: (e, 0)),
            pl.BlockSpec((1, D, FC), lambda e, f: (e, 0, f)),
            pl.BlockSpec((1, 1, FC), lambda e, f: (e, 0, f)),
            pl.BlockSpec((1, FC, D), lambda e, f: (e, f, 0)),
            pl.BlockSpec((1, 1, D), lambda e, f: (e, 0, 0)),
            pl.BlockSpec((CAP, E), lambda e, f: (0, 0)),
        ],
        out_specs=pl.BlockSpec((CAP, D), lambda e, f: (e, 0)),
        out_shape=jax.ShapeDtypeStruct((NSLOT, D), jnp.float32),
        scratch_shapes=[
            pltpu.VMEM((CAP, D), jnp.bfloat16),
            pltpu.VMEM((CAP, D), jnp.float32),
        ],
    )(disp, w1, b1.reshape(E, 1, F), w2, b2.reshape(E, 1, D), sw)

    y = pl.kernel(
        _sc_combine,
        mesh=mesh,
        out_type=jax.ShapeDtypeStruct((T, D), jnp.float32),
        scratch_types=[
            pltpu.VMEM((CH, D), jnp.float32),
            pltpu.VMEM((CH, D), jnp.float32),
            pltpu.VMEM((CH,), jnp.int32),
            pltpu.VMEM((CH,), jnp.int32),
            pltpu.SemaphoreType.DMA,
        ],
    )(eout, g0.reshape(T), g1.reshape(T))
    return y


# speedup vs baseline: 1.0895x; 1.0359x over previous
import functools

import jax
import jax.numpy as jnp
from jax import lax
from jax.experimental import pallas as pl
from jax.experimental.pallas import tpu as pltpu
from jax.experimental.pallas import tpu_sc as plsc

T = 2048
D = 1024
F = 4096
E = 8
K = 2
CAP = int(T * K / E * 1.25)

NF = 4
FC = F // NF

NSLOT = E * CAP
TRASH = NSLOT
NROW = NSLOT + 8

NW = 32
TPW = T // NW
CH = 16
NSUB = TPW // CH


def _router_kernel(x_ref, ls_ref, lb_ref, rw_ref,
                   hb_ref, s0_ref, s1_ref, g0_ref, g1_ref, sw_ref):
    x = x_ref[...]
    mu = jnp.mean(x, axis=-1, keepdims=True)
    xc = x - mu
    var = jnp.mean(xc * xc, axis=-1, keepdims=True)
    h = xc / jnp.sqrt(var + 1e-5) * ls_ref[...][None, :] + lb_ref[...][None, :]
    hb_ref[...] = h
    logits = jnp.dot(h, rw_ref[...], preferred_element_type=jnp.float32)
    mx = jnp.max(logits, axis=-1, keepdims=True)
    ex = jnp.exp(logits - mx)
    probs = ex / jnp.sum(ex, axis=-1, keepdims=True)
    iota = jax.lax.broadcasted_iota(jnp.int32, probs.shape, 1)
    v1 = jnp.max(probs, axis=-1, keepdims=True)
    i1 = jnp.min(jnp.where(probs == v1, iota, E), axis=-1, keepdims=True)
    masked = jnp.where(iota == i1, -jnp.inf, probs)
    v2 = jnp.max(masked, axis=-1, keepdims=True)
    i2 = jnp.min(jnp.where(masked == v2, iota, E), axis=-1, keepdims=True)
    s = v1 + v2 + 1e-8
    gv = jnp.concatenate([v1 / s, v2 / s], axis=-1)

    eiota = jax.lax.broadcasted_iota(jnp.int32, (T, E), 1)
    eoh0 = (i1 == eiota).astype(jnp.float32)
    eoh1 = (i2 == eiota).astype(jnp.float32)
    c = eoh0 + eoh1
    r = jax.lax.broadcasted_iota(jnp.int32, (T, T), 0)
    col = jax.lax.broadcasted_iota(jnp.int32, (T, T), 1)
    L = (col < r).astype(jnp.float32)
    excl = jax.lax.dot_general(
        L, c, (((1,), (0,)), ((), ())),
        preferred_element_type=jnp.float32,
        precision=jax.lax.Precision.HIGHEST)
    pos0 = jnp.sum(excl * eoh0, axis=-1, keepdims=True)
    pos1 = jnp.sum(excl * eoh1, axis=-1, keepdims=True)
    keep0 = pos0 < CAP
    keep1 = pos1 < CAP
    slot0 = i1 * CAP + pos0.astype(jnp.int32)
    slot1 = i2 * CAP + pos1.astype(jnp.int32)
    s0_ref[...] = jnp.where(keep0, slot0, TRASH)
    s1_ref[...] = jnp.where(keep1, slot1, TRASH)

    tot = excl[T - 1:T, :] + c[T - 1:T, :]
    tmin = jnp.min(tot, axis=-1, keepdims=True)
    e_iota_row = jax.lax.broadcasted_iota(jnp.int32, (1, E), 1)
    pe = jnp.min(jnp.where(tot == tmin, e_iota_row, E), axis=-1,
                 keepdims=True)
    pad_slot = pe * CAP + (CAP - 1)
    g0_ref[...] = jnp.where(keep0, slot0, pad_slot)
    g1_ref[...] = jnp.where(keep1, slot1, pad_slot)

    ident = (r == col).astype(jnp.float32)
    p01 = jnp.concatenate([pos0, pos1], axis=-1)
    p01_row = jax.lax.dot_general(
        p01, ident, (((0,), (0,)), ((), ())),
        preferred_element_type=jnp.float32,
        precision=jax.lax.Precision.HIGHEST)
    p01i = p01_row.astype(jnp.int32)
    citer = jax.lax.broadcasted_iota(jnp.int32, (CAP, T), 0)
    P0 = p01i[0:1, :] == citer
    P1 = p01i[1:2, :] == citer
    w = gv * jnp.concatenate(
        [keep0.astype(jnp.float32), keep1.astype(jnp.float32)], axis=-1)
    W0 = eoh0 * w[:, 0:1]
    W1 = eoh1 * w[:, 1:2]
    sw_ref[...] = (
        jnp.dot(P0.astype(jnp.float32), W0, preferred_element_type=jnp.float32,
                precision=jax.lax.Precision.HIGHEST)
        + jnp.dot(P1.astype(jnp.float32), W1, preferred_element_type=jnp.float32,
                  precision=jax.lax.Precision.HIGHEST))


def _sc_dispatch(h_hbm, s0_hbm, s1_hbm, disp_hbm, rows_v, i0_v, i1_v, sem):
    wid = lax.axis_index("s") * 2 + lax.axis_index("c")
    base = wid * TPW
    pltpu.sync_copy(h_hbm.at[pl.ds(base, TPW)], rows_v)
    pltpu.sync_copy(s0_hbm.at[pl.ds(base, TPW)], i0_v)
    pltpu.sync_copy(s1_hbm.at[pl.ds(base, TPW)], i1_v)
    c0 = pltpu.async_copy(rows_v, disp_hbm.at[i0_v], sem)
    c1 = pltpu.async_copy(rows_v, disp_hbm.at[i1_v], sem)
    c0.wait()
    c1.wait()


def _sc_combine(eout_hbm, g0_hbm, g1_hbm, y_hbm,
                a0_v, b0_v, a1_v, b1_v, i00_v, i01_v, i10_v, i11_v,
                sem0, sem1):
    wid = lax.axis_index("s") * 2 + lax.axis_index("c")
    base = wid * TPW
    abufs = (a0_v, a1_v)
    bbufs = (b0_v, b1_v)
    i0bufs = (i00_v, i10_v)
    i1bufs = (i01_v, i11_v)
    sems = (sem0, sem1)

    def start(sub):
        slot = sub % 2
        off = base + sub * CH
        pltpu.sync_copy(g0_hbm.at[pl.ds(off, CH)], i0bufs[slot])
        pltpu.sync_copy(g1_hbm.at[pl.ds(off, CH)], i1bufs[slot])
        ca = pltpu.async_copy(eout_hbm.at[i0bufs[slot]], abufs[slot],
                              sems[slot])
        cb = pltpu.async_copy(eout_hbm.at[i1bufs[slot]], bbufs[slot],
                              sems[slot])
        return ca, cb

    pend = start(0)
    for sub in range(NSUB):
        nxt = start(sub + 1) if sub + 1 < NSUB else None
        pend[0].wait()
        pend[1].wait()
        slot = sub % 2
        a_v, b_v = abufs[slot], bbufs[slot]

        def row_body(rr, carry):
            for cc in range(0, D, 16):
                a_v[rr, pl.ds(cc, 16)] = (a_v[rr, pl.ds(cc, 16)]
                                          + b_v[rr, pl.ds(cc, 16)])
            return carry

        lax.fori_loop(0, CH, row_body, 0)
        pltpu.sync_copy(a_v, y_hbm.at[pl.ds(base + sub * CH, CH)])
        pend = nxt


def _moe_kernel(disp_ref, w1_ref, b1_ref, w2_ref, b2_ref, sw_ref,
                eout_ref, dispb_ref, acc_ref):
    f = pl.program_id(1)

    @pl.when(f == 0)
    def _sanitize():
        d = disp_ref[...]
        dispb_ref[...] = jnp.where(jnp.isfinite(d), d, 0.0).astype(jnp.bfloat16)

    hidden = jnp.maximum(
        jnp.dot(dispb_ref[...], w1_ref[0].astype(jnp.bfloat16),
                preferred_element_type=jnp.float32)
        + b1_ref[0], 0.0).astype(jnp.bfloat16)
    part = jnp.dot(hidden, w2_ref[0].astype(jnp.bfloat16),
                   preferred_element_type=jnp.float32)

    @pl.when(f == 0)
    def _init_acc():
        acc_ref[...] = part

    @pl.when((f != 0) & (f != NF - 1))
    def _add_acc():
        acc_ref[...] = acc_ref[...] + part

    @pl.when(f == NF - 1)
    def _scale_out():
        e = pl.program_id(0)
        laneiota = jax.lax.broadcasted_iota(jnp.int32, (CAP, E), 1)
        swc = jnp.sum(jnp.where(laneiota == e, sw_ref[...], 0.0),
                      axis=-1, keepdims=True)
        eout_ref[...] = (acc_ref[...] + part + b2_ref[0]) * swc


def kernel(x, ln_scale, ln_bias, router_w, w1, b1, w2, b2):
    hb, s0, s1, g0, g1, sw = pl.pallas_call(
        _router_kernel,
        out_shape=[
            jax.ShapeDtypeStruct((T, D), jnp.float32),
            jax.ShapeDtypeStruct((T, 1), jnp.int32),
            jax.ShapeDtypeStruct((T, 1), jnp.int32),
            jax.ShapeDtypeStruct((T, 1), jnp.int32),
            jax.ShapeDtypeStruct((T, 1), jnp.int32),
            jax.ShapeDtypeStruct((CAP, E), jnp.float32),
        ],
    )(x, ln_scale, ln_bias, router_w)

    mesh = plsc.VectorSubcoreMesh(core_axis_name="c", subcore_axis_name="s")

    disp = pl.kernel(
        _sc_dispatch,
        mesh=mesh,
        out_type=jax.ShapeDtypeStruct((NROW, D), jnp.float32),
        scratch_types=[
            pltpu.VMEM((TPW, D), jnp.float32),
            pltpu.VMEM((TPW,), jnp.int32),
            pltpu.VMEM((TPW,), jnp.int32),
            pltpu.SemaphoreType.DMA,
        ],
    )(hb, s0.reshape(T), s1.reshape(T))

    eout = pl.pallas_call(
        _moe_kernel,
        grid=(E, NF),
        in_specs=[
            pl.BlockSpec((CAP, D), lambda e, f: (e, 0)),
            pl.BlockSpec((1, D, FC), lambda e, f: (e, 0, f)),
            pl.BlockSpec((1, 1, FC), lambda e, f: (e, 0, f)),
            pl.BlockSpec((1, FC, D), lambda e, f: (e, f, 0)),
            pl.BlockSpec((1, 1, D), lambda e, f: (e, 0, 0)),
            pl.BlockSpec((CAP, E), lambda e, f: (0, 0)),
        ],
        out_specs=pl.BlockSpec((CAP, D), lambda e, f: (e, 0)),
        out_shape=jax.ShapeDtypeStruct((NSLOT, D), jnp.float32),
        scratch_shapes=[
            pltpu.VMEM((CAP, D), jnp.bfloat16),
            pltpu.VMEM((CAP, D), jnp.float32),
        ],
    )(disp, w1, b1.reshape(E, 1, F), w2,
      b2.reshape(E, 1, D), sw)

    y = pl.kernel(
        _sc_combine,
        mesh=mesh,
        out_type=jax.ShapeDtypeStruct((T, D), jnp.float32),
        scratch_types=[
            pltpu.VMEM((CH, D), jnp.float32),
            pltpu.VMEM((CH, D), jnp.float32),
            pltpu.VMEM((CH, D), jnp.float32),
            pltpu.VMEM((CH, D), jnp.float32),
            pltpu.VMEM((CH,), jnp.int32),
            pltpu.VMEM((CH,), jnp.int32),
            pltpu.VMEM((CH,), jnp.int32),
            pltpu.VMEM((CH,), jnp.int32),
            pltpu.SemaphoreType.DMA,
            pltpu.SemaphoreType.DMA,
        ],
    )(eout, g0.reshape(T), g1.reshape(T))
    return y


# speedup vs baseline: 1.2856x; 1.1800x over previous
import functools

import jax
import jax.numpy as jnp
from jax import lax
from jax.experimental import pallas as pl
from jax.experimental.pallas import tpu as pltpu
from jax.experimental.pallas import tpu_sc as plsc

T = 2048
D = 1024
F = 4096
E = 8
K = 2
CAP = int(T * K / E * 1.25)

NF = 4
FC = F // NF

NSLOT = E * CAP
TRASH = NSLOT
NROW = NSLOT + 8

NW = 32
TPW = T // NW
CH = 16
NSUB = TPW // CH


def _router_kernel(x_ref, ls_ref, lb_ref, rw_ref,
                   hb_ref, s0_ref, s1_ref, g0_ref, g1_ref, sw_ref):
    x = x_ref[...]
    mu = jnp.mean(x, axis=-1, keepdims=True)
    xc = x - mu
    var = jnp.mean(xc * xc, axis=-1, keepdims=True)
    h = xc / jnp.sqrt(var + 1e-5) * ls_ref[...][None, :] + lb_ref[...][None, :]
    hb_ref[...] = h
    logits = jnp.dot(h, rw_ref[...], preferred_element_type=jnp.float32)
    mx = jnp.max(logits, axis=-1, keepdims=True)
    ex = jnp.exp(logits - mx)
    probs = ex / jnp.sum(ex, axis=-1, keepdims=True)
    iota = jax.lax.broadcasted_iota(jnp.int32, probs.shape, 1)
    v1 = jnp.max(probs, axis=-1, keepdims=True)
    i1 = jnp.min(jnp.where(probs == v1, iota, E), axis=-1, keepdims=True)
    masked = jnp.where(iota == i1, -jnp.inf, probs)
    v2 = jnp.max(masked, axis=-1, keepdims=True)
    i2 = jnp.min(jnp.where(masked == v2, iota, E), axis=-1, keepdims=True)
    s = v1 + v2 + 1e-8
    gv = jnp.concatenate([v1 / s, v2 / s], axis=-1)

    eiota = jax.lax.broadcasted_iota(jnp.int32, (T, E), 1)
    eoh0 = (i1 == eiota).astype(jnp.float32)
    eoh1 = (i2 == eiota).astype(jnp.float32)
    c = eoh0 + eoh1
    r = jax.lax.broadcasted_iota(jnp.int32, (T, T), 0)
    col = jax.lax.broadcasted_iota(jnp.int32, (T, T), 1)
    L = (col < r).astype(jnp.float32)
    excl = jax.lax.dot_general(
        L, c, (((1,), (0,)), ((), ())),
        preferred_element_type=jnp.float32)
    pos0 = jnp.sum(excl * eoh0, axis=-1, keepdims=True)
    pos1 = jnp.sum(excl * eoh1, axis=-1, keepdims=True)
    keep0 = pos0 < CAP
    keep1 = pos1 < CAP
    slot0 = i1 * CAP + pos0.astype(jnp.int32)
    slot1 = i2 * CAP + pos1.astype(jnp.int32)
    s0_ref[...] = jnp.where(keep0, slot0, TRASH)
    s1_ref[...] = jnp.where(keep1, slot1, TRASH)

    tot = excl[T - 1:T, :] + c[T - 1:T, :]
    tmin = jnp.min(tot, axis=-1, keepdims=True)
    e_iota_row = jax.lax.broadcasted_iota(jnp.int32, (1, E), 1)
    pe = jnp.min(jnp.where(tot == tmin, e_iota_row, E), axis=-1,
                 keepdims=True)
    pad_slot = pe * CAP + (CAP - 1)
    g0_ref[...] = jnp.where(keep0, slot0, pad_slot)
    g1_ref[...] = jnp.where(keep1, slot1, pad_slot)

    ident = (r == col).astype(jnp.float32)
    p01 = jnp.concatenate([pos0, pos1], axis=-1)
    p01_row = jax.lax.dot_general(
        p01, ident, (((0,), (0,)), ((), ())),
        preferred_element_type=jnp.float32,
        precision=jax.lax.Precision.HIGHEST)
    p01i = p01_row.astype(jnp.int32)
    citer = jax.lax.broadcasted_iota(jnp.int32, (CAP, T), 0)
    P0 = p01i[0:1, :] == citer
    P1 = p01i[1:2, :] == citer
    w = gv * jnp.concatenate(
        [keep0.astype(jnp.float32), keep1.astype(jnp.float32)], axis=-1)
    W0 = eoh0 * w[:, 0:1]
    W1 = eoh1 * w[:, 1:2]
    sw_ref[...] = (
        jnp.dot(P0.astype(jnp.float32), W0, preferred_element_type=jnp.float32)
        + jnp.dot(P1.astype(jnp.float32), W1,
                  preferred_element_type=jnp.float32))


def _sc_dispatch(h_hbm, s0_hbm, s1_hbm, disp_hbm, rows_v, i0_v, i1_v, sem):
    wid = lax.axis_index("s") * 2 + lax.axis_index("c")
    base = wid * TPW
    pltpu.sync_copy(h_hbm.at[pl.ds(base, TPW)], rows_v)
    pltpu.sync_copy(s0_hbm.at[pl.ds(base, TPW)], i0_v)
    pltpu.sync_copy(s1_hbm.at[pl.ds(base, TPW)], i1_v)
    c0 = pltpu.async_copy(rows_v, disp_hbm.at[i0_v], sem)
    c1 = pltpu.async_copy(rows_v, disp_hbm.at[i1_v], sem)
    c0.wait()
    c1.wait()


def _sc_combine(eout_hbm, g0_hbm, g1_hbm, y_hbm,
                a0_v, b0_v, a1_v, b1_v, i00_v, i01_v, i10_v, i11_v,
                sem0, sem1):
    wid = lax.axis_index("s") * 2 + lax.axis_index("c")
    base = wid * TPW
    abufs = (a0_v, a1_v)
    bbufs = (b0_v, b1_v)
    i0bufs = (i00_v, i10_v)
    i1bufs = (i01_v, i11_v)
    sems = (sem0, sem1)

    def start(sub):
        slot = sub % 2
        off = base + sub * CH
        pltpu.sync_copy(g0_hbm.at[pl.ds(off, CH)], i0bufs[slot])
        pltpu.sync_copy(g1_hbm.at[pl.ds(off, CH)], i1bufs[slot])
        ca = pltpu.async_copy(eout_hbm.at[i0bufs[slot]], abufs[slot],
                              sems[slot])
        cb = pltpu.async_copy(eout_hbm.at[i1bufs[slot]], bbufs[slot],
                              sems[slot])
        return ca, cb

    pend = start(0)
    for sub in range(NSUB):
        nxt = start(sub + 1) if sub + 1 < NSUB else None
        pend[0].wait()
        pend[1].wait()
        slot = sub % 2
        a_v, b_v = abufs[slot], bbufs[slot]

        def row_body(rr, carry):
            for cc in range(0, D, 16):
                a_v[rr, pl.ds(cc, 16)] = (a_v[rr, pl.ds(cc, 16)]
                                          + b_v[rr, pl.ds(cc, 16)])
            return carry

        lax.fori_loop(0, CH, row_body, 0)
        pltpu.sync_copy(a_v, y_hbm.at[pl.ds(base + sub * CH, CH)])
        pend = nxt


def _moe_kernel(disp_ref, w1_ref, b1_ref, w2_ref, b2_ref, sw_ref,
                eout_ref, dispb_ref, acc_ref):
    f = pl.program_id(1)

    @pl.when(f == 0)
    def _sanitize():
        d = disp_ref[...]
        dispb_ref[...] = jnp.where(jnp.isfinite(d), d, 0.0).astype(jnp.bfloat16)

    hidden = jnp.maximum(
        jnp.dot(dispb_ref[...], w1_ref[0].astype(jnp.bfloat16),
                preferred_element_type=jnp.float32)
        + b1_ref[0], 0.0).astype(jnp.bfloat16)
    part = jnp.dot(hidden, w2_ref[0].astype(jnp.bfloat16),
                   preferred_element_type=jnp.float32)

    @pl.when(f == 0)
    def _init_acc():
        acc_ref[...] = part

    @pl.when((f != 0) & (f != NF - 1))
    def _add_acc():
        acc_ref[...] = acc_ref[...] + part

    @pl.when(f == NF - 1)
    def _scale_out():
        e = pl.program_id(0)
        laneiota = jax.lax.broadcasted_iota(jnp.int32, (CAP, E), 1)
        swc = jnp.sum(jnp.where(laneiota == e, sw_ref[...], 0.0),
                      axis=-1, keepdims=True)
        eout_ref[...] = (acc_ref[...] + part + b2_ref[0]) * swc


def kernel(x, ln_scale, ln_bias, router_w, w1, b1, w2, b2):
    hb, s0, s1, g0, g1, sw = pl.pallas_call(
        _router_kernel,
        out_shape=[
            jax.ShapeDtypeStruct((T, D), jnp.float32),
            jax.ShapeDtypeStruct((T, 1), jnp.int32),
            jax.ShapeDtypeStruct((T, 1), jnp.int32),
            jax.ShapeDtypeStruct((T, 1), jnp.int32),
            jax.ShapeDtypeStruct((T, 1), jnp.int32),
            jax.ShapeDtypeStruct((CAP, E), jnp.float32),
        ],
    )(x, ln_scale, ln_bias, router_w)

    mesh = plsc.VectorSubcoreMesh(core_axis_name="c", subcore_axis_name="s")

    disp = pl.kernel(
        _sc_dispatch,
        mesh=mesh,
        out_type=jax.ShapeDtypeStruct((NROW, D), jnp.float32),
        scratch_types=[
            pltpu.VMEM((TPW, D), jnp.float32),
            pltpu.VMEM((TPW,), jnp.int32),
            pltpu.VMEM((TPW,), jnp.int32),
            pltpu.SemaphoreType.DMA,
        ],
    )(hb, s0.reshape(T), s1.reshape(T))

    eout = pl.pallas_call(
        _moe_kernel,
        grid=(E, NF),
        in_specs=[
            pl.BlockSpec((CAP, D), lambda e, f: (e, 0)),
            pl.BlockSpec((1, D, FC), lambda e, f: (e, 0, f)),
            pl.BlockSpec((1, 1, FC), lambda e, f: (e, 0, f)),
            pl.BlockSpec((1, FC, D), lambda e, f: (e, f, 0)),
            pl.BlockSpec((1, 1, D), lambda e, f: (e, 0, 0)),
            pl.BlockSpec((CAP, E), lambda e, f: (0, 0)),
        ],
        out_specs=pl.BlockSpec((CAP, D), lambda e, f: (e, 0)),
        out_shape=jax.ShapeDtypeStruct((NSLOT, D), jnp.float32),
        scratch_shapes=[
            pltpu.VMEM((CAP, D), jnp.bfloat16),
            pltpu.VMEM((CAP, D), jnp.float32),
        ],
    )(disp, w1, b1.reshape(E, 1, F), w2,
      b2.reshape(E, 1, D), sw)

    y = pl.kernel(
        _sc_combine,
        mesh=mesh,
        out_type=jax.ShapeDtypeStruct((T, D), jnp.float32),
        scratch_types=[
            pltpu.VMEM((CH, D), jnp.float32),
            pltpu.VMEM((CH, D), jnp.float32),
            pltpu.VMEM((CH, D), jnp.float32),
            pltpu.VMEM((CH, D), jnp.float32),
            pltpu.VMEM((CH,), jnp.int32),
            pltpu.VMEM((CH,), jnp.int32),
            pltpu.VMEM((CH,), jnp.int32),
            pltpu.VMEM((CH,), jnp.int32),
            pltpu.SemaphoreType.DMA,
            pltpu.SemaphoreType.DMA,
        ],
    )(eout, g0.reshape(T), g1.reshape(T))
    return y


# speedup vs baseline: 1.2896x; 1.0031x over previous
import functools

import jax
import jax.numpy as jnp
from jax import lax
from jax.experimental import pallas as pl
from jax.experimental.pallas import tpu as pltpu
from jax.experimental.pallas import tpu_sc as plsc

T = 2048
D = 1024
F = 4096
E = 8
K = 2
CAP = int(T * K / E * 1.25)

NF = 4
FC = F // NF

NSLOT = E * CAP
TRASH = NSLOT
NROW = NSLOT + 8

NW = 32
TPW = T // NW
CH = 16
NSUB = TPW // CH


def _router_kernel(x_ref, ls_ref, lb_ref, rw_ref,
                   hb_ref, s0_ref, s1_ref, g0_ref, g1_ref, sw_ref):
    x = x_ref[...]
    mu = jnp.mean(x, axis=-1, keepdims=True)
    xc = x - mu
    var = jnp.mean(xc * xc, axis=-1, keepdims=True)
    h = xc / jnp.sqrt(var + 1e-5) * ls_ref[...][None, :] + lb_ref[...][None, :]
    hb_ref[...] = h
    logits = jnp.dot(h, rw_ref[...], preferred_element_type=jnp.float32)
    mx = jnp.max(logits, axis=-1, keepdims=True)
    ex = jnp.exp(logits - mx)
    probs = ex / jnp.sum(ex, axis=-1, keepdims=True)
    iota = jax.lax.broadcasted_iota(jnp.int32, probs.shape, 1)
    v1 = jnp.max(probs, axis=-1, keepdims=True)
    i1 = jnp.min(jnp.where(probs == v1, iota, E), axis=-1, keepdims=True)
    masked = jnp.where(iota == i1, -jnp.inf, probs)
    v2 = jnp.max(masked, axis=-1, keepdims=True)
    i2 = jnp.min(jnp.where(masked == v2, iota, E), axis=-1, keepdims=True)
    s = v1 + v2 + 1e-8
    gv = jnp.concatenate([v1 / s, v2 / s], axis=-1)

    eiota = jax.lax.broadcasted_iota(jnp.int32, (T, E), 1)
    eoh0 = (i1 == eiota).astype(jnp.float32)
    eoh1 = (i2 == eiota).astype(jnp.float32)
    c = eoh0 + eoh1
    r = jax.lax.broadcasted_iota(jnp.int32, (T, T), 0)
    col = jax.lax.broadcasted_iota(jnp.int32, (T, T), 1)
    L = (col < r).astype(jnp.float32)
    excl = jax.lax.dot_general(
        L, c, (((1,), (0,)), ((), ())),
        preferred_element_type=jnp.float32)
    pos0 = jnp.sum(excl * eoh0, axis=-1, keepdims=True)
    pos1 = jnp.sum(excl * eoh1, axis=-1, keepdims=True)
    keep0 = pos0 < CAP
    keep1 = pos1 < CAP
    slot0 = i1 * CAP + pos0.astype(jnp.int32)
    slot1 = i2 * CAP + pos1.astype(jnp.int32)
    s0_ref[...] = jnp.where(keep0, slot0, TRASH)
    s1_ref[...] = jnp.where(keep1, slot1, TRASH)

    tot = excl[T - 1:T, :] + c[T - 1:T, :]
    tmin = jnp.min(tot, axis=-1, keepdims=True)
    e_iota_row = jax.lax.broadcasted_iota(jnp.int32, (1, E), 1)
    pe = jnp.min(jnp.where(tot == tmin, e_iota_row, E), axis=-1,
                 keepdims=True)
    pad_slot = pe * CAP + (CAP - 1)
    g0_ref[...] = jnp.where(keep0, slot0, pad_slot)
    g1_ref[...] = jnp.where(keep1, slot1, pad_slot)

    p01 = jnp.concatenate([pos0, pos1], axis=-1)
    p01i = jnp.transpose(p01, (1, 0)).astype(jnp.int32)
    citer = jax.lax.broadcasted_iota(jnp.int32, (CAP, T), 0)
    P0 = p01i[0:1, :] == citer
    P1 = p01i[1:2, :] == citer
    w = gv * jnp.concatenate(
        [keep0.astype(jnp.float32), keep1.astype(jnp.float32)], axis=-1)
    W0 = eoh0 * w[:, 0:1]
    W1 = eoh1 * w[:, 1:2]
    sw_ref[...] = (
        jnp.dot(P0.astype(jnp.float32), W0, preferred_element_type=jnp.float32)
        + jnp.dot(P1.astype(jnp.float32), W1,
                  preferred_element_type=jnp.float32))


def _sc_dispatch(h_hbm, s0_hbm, s1_hbm, disp_hbm, rows_v, i0_v, i1_v, sem):
    wid = lax.axis_index("s") * 2 + lax.axis_index("c")
    base = wid * TPW
    pltpu.sync_copy(h_hbm.at[pl.ds(base, TPW)], rows_v)
    pltpu.sync_copy(s0_hbm.at[pl.ds(base, TPW)], i0_v)
    pltpu.sync_copy(s1_hbm.at[pl.ds(base, TPW)], i1_v)
    c0 = pltpu.async_copy(rows_v, disp_hbm.at[i0_v], sem)
    c1 = pltpu.async_copy(rows_v, disp_hbm.at[i1_v], sem)
    c0.wait()
    c1.wait()


def _sc_combine(eout_hbm, g0_hbm, g1_hbm, y_hbm,
                a0_v, b0_v, a1_v, b1_v, i00_v, i01_v, i10_v, i11_v,
                sem0, sem1):
    wid = lax.axis_index("s") * 2 + lax.axis_index("c")
    base = wid * TPW
    abufs = (a0_v, a1_v)
    bbufs = (b0_v, b1_v)
    i0bufs = (i00_v, i10_v)
    i1bufs = (i01_v, i11_v)
    sems = (sem0, sem1)

    def start(sub):
        slot = sub % 2
        off = base + sub * CH
        pltpu.sync_copy(g0_hbm.at[pl.ds(off, CH)], i0bufs[slot])
        pltpu.sync_copy(g1_hbm.at[pl.ds(off, CH)], i1bufs[slot])
        ca = pltpu.async_copy(eout_hbm.at[i0bufs[slot]], abufs[slot],
                              sems[slot])
        cb = pltpu.async_copy(eout_hbm.at[i1bufs[slot]], bbufs[slot],
                              sems[slot])
        return ca, cb

    pend = start(0)
    for sub in range(NSUB):
        nxt = start(sub + 1) if sub + 1 < NSUB else None
        pend[0].wait()
        pend[1].wait()
        slot = sub % 2
        a_v, b_v = abufs[slot], bbufs[slot]

        def row_body(rr, carry):
            for cc in range(0, D, 16):
                a_v[rr, pl.ds(cc, 16)] = (a_v[rr, pl.ds(cc, 16)]
                                          + b_v[rr, pl.ds(cc, 16)])
            return carry

        lax.fori_loop(0, CH, row_body, 0)
        pltpu.sync_copy(a_v, y_hbm.at[pl.ds(base + sub * CH, CH)])
        pend = nxt


def _moe_kernel(disp_ref, w1_ref, b1_ref, w2_ref, b2_ref, sw_ref,
                eout_ref, dispb_ref, acc_ref):
    f = pl.program_id(1)

    @pl.when(f == 0)
    def _sanitize():
        d = disp_ref[...]
        dispb_ref[...] = jnp.where(jnp.isfinite(d), d, 0.0).astype(jnp.bfloat16)

    hidden = jnp.maximum(
        jnp.dot(dispb_ref[...], w1_ref[0].astype(jnp.bfloat16),
                preferred_element_type=jnp.float32)
        + b1_ref[0], 0.0).astype(jnp.bfloat16)
    part = jnp.dot(hidden, w2_ref[0].astype(jnp.bfloat16),
                   preferred_element_type=jnp.float32)

    @pl.when(f == 0)
    def _init_acc():
        acc_ref[...] = part

    @pl.when((f != 0) & (f != NF - 1))
    def _add_acc():
        acc_ref[...] = acc_ref[...] + part

    @pl.when(f == NF - 1)
    def _scale_out():
        e = pl.program_id(0)
        laneiota = jax.lax.broadcasted_iota(jnp.int32, (CAP, E), 1)
        swc = jnp.sum(jnp.where(laneiota == e, sw_ref[...], 0.0),
                      axis=-1, keepdims=True)
        eout_ref[...] = (acc_ref[...] + part + b2_ref[0]) * swc


def kernel(x, ln_scale, ln_bias, router_w, w1, b1, w2, b2):
    hb, s0, s1, g0, g1, sw = pl.pallas_call(
        _router_kernel,
        out_shape=[
            jax.ShapeDtypeStruct((T, D), jnp.float32),
            jax.ShapeDtypeStruct((T, 1), jnp.int32),
            jax.ShapeDtypeStruct((T, 1), jnp.int32),
            jax.ShapeDtypeStruct((T, 1), jnp.int32),
            jax.ShapeDtypeStruct((T, 1), jnp.int32),
            jax.ShapeDtypeStruct((CAP, E), jnp.float32),
        ],
    )(x, ln_scale, ln_bias, router_w)

    mesh = plsc.VectorSubcoreMesh(core_axis_name="c", subcore_axis_name="s")

    disp = pl.kernel(
        _sc_dispatch,
        mesh=mesh,
        out_type=jax.ShapeDtypeStruct((NROW, D), jnp.float32),
        scratch_types=[
            pltpu.VMEM((TPW, D), jnp.float32),
            pltpu.VMEM((TPW,), jnp.int32),
            pltpu.VMEM((TPW,), jnp.int32),
            pltpu.SemaphoreType.DMA,
        ],
    )(hb, s0.reshape(T), s1.reshape(T))

    eout = pl.pallas_call(
        _moe_kernel,
        grid=(E, NF),
        in_specs=[
            pl.BlockSpec((CAP, D), lambda e, f: (e, 0)),
            pl.BlockSpec((1, D, FC), lambda e, f: (e, 0, f)),
            pl.BlockSpec((1, 1, FC), lambda e, f: (e, 0, f)),
            pl.BlockSpec((1, FC, D), lambda e, f: (e, f, 0)),
            pl.BlockSpec((1, 1, D), lambda e, f: (e, 0, 0)),
            pl.BlockSpec((CAP, E), lambda e, f: (0, 0)),
        ],
        out_specs=pl.BlockSpec((CAP, D), lambda e, f: (e, 0)),
        out_shape=jax.ShapeDtypeStruct((NSLOT, D), jnp.float32),
        scratch_shapes=[
            pltpu.VMEM((CAP, D), jnp.bfloat16),
            pltpu.VMEM((CAP, D), jnp.float32),
        ],
    )(disp, w1, b1.reshape(E, 1, F), w2,
      b2.reshape(E, 1, D), sw)

    y = pl.kernel(
        _sc_combine,
        mesh=mesh,
        out_type=jax.ShapeDtypeStruct((T, D), jnp.float32),
        scratch_types=[
            pltpu.VMEM((CH, D), jnp.float32),
            pltpu.VMEM((CH, D), jnp.float32),
            pltpu.VMEM((CH, D), jnp.float32),
            pltpu.VMEM((CH, D), jnp.float32),
            pltpu.VMEM((CH,), jnp.int32),
            pltpu.VMEM((CH,), jnp.int32),
            pltpu.VMEM((CH,), jnp.int32),
            pltpu.VMEM((CH,), jnp.int32),
            pltpu.SemaphoreType.DMA,
            pltpu.SemaphoreType.DMA,
        ],
    )(eout, g0.reshape(T), g1.reshape(T))
    return y
